# baseline (speedup 1.0000x reference)
import functools

import jax
import jax.numpy as jnp
from jax import lax
from jax.experimental import pallas as pl
from jax.experimental.pallas import tpu as pltpu
from jax.experimental.pallas import tpu_sc as plsc

N_CB = 8
CB_SIZE = 2048
D = 1024
B = 4
T = 4096

NUM_CORES = 2
NUM_SUBCORES = 16
NUM_WORKERS = NUM_CORES * NUM_SUBCORES
ROWS_PER_W = (B * T) // NUM_WORKERS
CHUNK = 8
N_CHUNKS = ROWS_PER_W // CHUNK
N_PAIRS = N_CHUNKS // 2
VREGS_PER_ROW = D // 16


def _body(x_hbm, tab_hbm, out_hbm, idx_v, bufs, st0_v, st1_v,
          sgs, so0, so1):
    wid = lax.axis_index("s") * NUM_CORES + lax.axis_index("c")
    tpw = T // (NUM_WORKERS // B)
    b = wid // (NUM_WORKERS // B)
    t0 = (wid % (NUM_WORKERS // B)) * tpw
    wbase = wid * ROWS_PER_W

    pltpu.sync_copy(x_hbm.at[b, :, pl.ds(t0, tpw)], idx_v)

    @pl.loop(0, tpw // 16)
    def _offsets(j):
        sl = pl.ds(j * 16, 16)
        for i in range(1, N_CB):
            idx_v[i, sl] = idx_v[i, sl] + i * CB_SIZE

    def gather(cb, r0, dst, sem):
        pltpu.async_copy(tab_hbm.at[idx_v.at[cb, pl.ds(r0, CHUNK)]], dst, sem)

    def wait_gather(dst, sem):
        pltpu.make_async_copy(
            tab_hbm.at[idx_v.at[0, pl.ds(0, CHUNK)]], dst, sem).wait()

    def wait_out(st, sem):
        pltpu.make_async_copy(st, out_hbm.at[pl.ds(0, CHUNK)], sem).wait()

    def tree_sum(st):
        @plsc.parallel_loop(0, CHUNK * VREGS_PER_ROW, 1, unroll=4)
        def _vregs(j):
            r = j // VREGS_PER_ROW
            c = (j % VREGS_PER_ROW) * 16
            sl = pl.ds(c, 16)
            v01 = bufs[0][r, sl] + bufs[1][r, sl]
            v23 = bufs[2][r, sl] + bufs[3][r, sl]
            v45 = bufs[4][r, sl] + bufs[5][r, sl]
            v67 = bufs[6][r, sl] + bufs[7][r, sl]
            st[r, sl] = (v01 + v23) + (v45 + v67)

    def do_chunk(r0, st, so, out_wait_cond, prefetch_cond):
        for k in range(N_CB):
            wait_gather(bufs[k], sgs[k])

        @pl.when(out_wait_cond)
        def _drain_prev_out():
            wait_out(st, so)

        tree_sum(st)

        @pl.when(prefetch_cond)
        def _prefetch():
            for k in range(N_CB):
                gather(k, r0 + CHUNK, bufs[k], sgs[k])

        pltpu.async_copy(st, out_hbm.at[pl.ds(wbase + r0, CHUNK)], so)

    for k in range(N_CB):
        gather(k, 0, bufs[k], sgs[k])

    true_ = jnp.bool_(True)

    @pl.loop(0, N_PAIRS)
    def _pair(j):
        r0 = 2 * j * CHUNK
        do_chunk(r0, st0_v, so0, out_wait_cond=j > 0, prefetch_cond=true_)
        do_chunk(r0 + CHUNK, st1_v, so1, out_wait_cond=j > 0,
                 prefetch_cond=j < N_PAIRS - 1)

    wait_out(st0_v, so0)
    wait_out(st1_v, so1)


@jax.jit
def _run(x, tables):
    tab_flat = tables.reshape(N_CB * CB_SIZE, D)
    mesh = plsc.VectorSubcoreMesh(core_axis_name="c", subcore_axis_name="s")
    call = pl.kernel(
        lambda x_hbm, tab_hbm, out_hbm, idx_v, b0, b1, b2, b3, b4, b5, b6,
               b7, st0_v, st1_v, sg0, sg1, sg2, sg3, sg4, sg5, sg6, sg7,
               so0, so1:
            _body(x_hbm, tab_hbm, out_hbm, idx_v,
                  (b0, b1, b2, b3, b4, b5, b6, b7), st0_v, st1_v,
                  (sg0, sg1, sg2, sg3, sg4, sg5, sg6, sg7), so0, so1),
        out_type=jax.ShapeDtypeStruct((B * T, D), jnp.float32),
        mesh=mesh,
        scratch_types=[
            pltpu.VMEM((N_CB, ROWS_PER_W), jnp.int32),
            *[pltpu.VMEM((CHUNK, D), jnp.float32) for _ in range(N_CB)],
            pltpu.VMEM((CHUNK, D), jnp.float32),
            pltpu.VMEM((CHUNK, D), jnp.float32),
            *[pltpu.SemaphoreType.DMA for _ in range(N_CB)],
            pltpu.SemaphoreType.DMA,
            pltpu.SemaphoreType.DMA,
        ],
    )
    out_flat = call(x, tab_flat)
    return out_flat.reshape(B, T, D)


def kernel(x, tables):
    return _run(x.astype(jnp.int32), tables)


# speedup vs baseline: 1.3428x; 1.3428x over previous
import functools

import jax
import jax.numpy as jnp
from jax import lax
from jax.experimental import pallas as pl
from jax.experimental.pallas import tpu as pltpu
from jax.experimental.pallas import tpu_sc as plsc

N_CB = 8
CB_SIZE = 2048
D = 1024
B = 4
T = 4096

NUM_CORES = 2
NUM_SUBCORES = 16
NUM_WORKERS = NUM_CORES * NUM_SUBCORES
ROWS_PER_W = (B * T) // NUM_WORKERS
CHUNK = 16
N_CHUNKS = ROWS_PER_W // CHUNK
N_PAIRS = N_CHUNKS // 2
VREGS_PER_ROW = D // 16


def _body(x_hbm, tab_hbm, out_hbm, idx_v, acc0_v, acc1_v,
          bufa_v, bufb_v, bufc_v, bufd_v,
          sa0, sa1, sba, sbb, sbc, sbd, so0, so1):
    wid = lax.axis_index("s") * NUM_CORES + lax.axis_index("c")
    tpw = T // (NUM_WORKERS // B)
    b = wid // (NUM_WORKERS // B)
    t0 = (wid % (NUM_WORKERS // B)) * tpw
    wbase = wid * ROWS_PER_W

    pltpu.sync_copy(x_hbm.at[b, :, pl.ds(t0, tpw)], idx_v)

    @pl.loop(0, tpw // 16)
    def _offsets(j):
        sl = pl.ds(j * 16, 16)
        for i in range(1, N_CB):
            idx_v[i, sl] = idx_v[i, sl] + i * CB_SIZE

    def gather(cb, r0, dst, sem):
        pltpu.async_copy(tab_hbm.at[idx_v.at[cb, pl.ds(r0, CHUNK)]], dst, sem)

    def wait_gather(dst, sem):
        pltpu.make_async_copy(
            tab_hbm.at[idx_v.at[0, pl.ds(0, CHUNK)]], dst, sem).wait()

    def out_write(acc, r0, sem):
        pltpu.async_copy(acc, out_hbm.at[pl.ds(wbase + r0, CHUNK)], sem)

    def wait_out(acc, sem):
        pltpu.make_async_copy(acc, out_hbm.at[pl.ds(0, CHUNK)], sem).wait()

    def acc_pair(acc, u, v):
        @plsc.parallel_loop(0, CHUNK * VREGS_PER_ROW, 1, unroll=8)
        def _vregs(j):
            r = j // VREGS_PER_ROW
            sl = pl.ds((j % VREGS_PER_ROW) * 16, 16)
            plsc.addupdate(acc.at[r, sl], u[r, sl] + v[r, sl])

    def acc_one(acc, u):
        @plsc.parallel_loop(0, CHUNK * VREGS_PER_ROW, 1, unroll=16)
        def _vregs(j):
            r = j // VREGS_PER_ROW
            sl = pl.ds((j % VREGS_PER_ROW) * 16, 16)
            plsc.addupdate(acc.at[r, sl], u[r, sl])

    def do_chunk(r0, acc, sa, so, acc_o, sa_o, so_o,
                 out_wait_cond, prefetch_cond):
        r_next = r0 + CHUNK

        wait_gather(acc, sa)
        wait_gather(bufa_v, sba)
        wait_gather(bufb_v, sbb)
        acc_pair(acc, bufa_v, bufb_v)
        gather(5, r0, bufa_v, sba)
        gather(6, r0, bufb_v, sbb)

        @pl.when(out_wait_cond)
        def _drain_prev_out():
            wait_out(acc_o, so_o)

        @pl.when(prefetch_cond)
        def _pf0():
            gather(0, r_next, acc_o, sa_o)

        wait_gather(bufc_v, sbc)
        wait_gather(bufd_v, sbd)
        acc_pair(acc, bufc_v, bufd_v)
        gather(7, r0, bufc_v, sbc)

        @pl.when(prefetch_cond)
        def _pf4():
            gather(4, r_next, bufd_v, sbd)

        wait_gather(bufa_v, sba)
        wait_gather(bufb_v, sbb)
        acc_pair(acc, bufa_v, bufb_v)

        @pl.when(prefetch_cond)
        def _pf12():
            gather(1, r_next, bufa_v, sba)
            gather(2, r_next, bufb_v, sbb)

        wait_gather(bufc_v, sbc)
        acc_one(acc, bufc_v)

        @pl.when(prefetch_cond)
        def _pf3():
            gather(3, r_next, bufc_v, sbc)

        out_write(acc, r0, so)

    gather(0, 0, acc0_v, sa0)
    gather(1, 0, bufa_v, sba)
    gather(2, 0, bufb_v, sbb)
    gather(3, 0, bufc_v, sbc)
    gather(4, 0, bufd_v, sbd)

    true_ = jnp.bool_(True)

    @pl.loop(0, N_PAIRS)
    def _pair(j):
        r0 = 2 * j * CHUNK
        do_chunk(r0, acc0_v, sa0, so0, acc1_v, sa1, so1,
                 out_wait_cond=j > 0, prefetch_cond=true_)
        do_chunk(r0 + CHUNK, acc1_v, sa1, so1, acc0_v, sa0, so0,
                 out_wait_cond=true_, prefetch_cond=j < N_PAIRS - 1)

    wait_out(acc1_v, so1)


@jax.jit
def _run(x, tables):
    tab_flat = tables.reshape(N_CB * CB_SIZE, D)
    mesh = plsc.VectorSubcoreMesh(core_axis_name="c", subcore_axis_name="s")
    call = pl.kernel(
        _body,
        out_type=jax.ShapeDtypeStruct((B * T, D), jnp.float32),
        mesh=mesh,
        scratch_types=[
            pltpu.VMEM((N_CB, ROWS_PER_W), jnp.int32),
            pltpu.VMEM((CHUNK, D), jnp.float32),
            pltpu.VMEM((CHUNK, D), jnp.float32),
            pltpu.VMEM((CHUNK, D), jnp.float32),
            pltpu.VMEM((CHUNK, D), jnp.float32),
            pltpu.VMEM((CHUNK, D), jnp.float32),
            pltpu.VMEM((CHUNK, D), jnp.float32),
            pltpu.SemaphoreType.DMA,
            pltpu.SemaphoreType.DMA,
            pltpu.SemaphoreType.DMA,
            pltpu.SemaphoreType.DMA,
            pltpu.SemaphoreType.DMA,
            pltpu.SemaphoreType.DMA,
            pltpu.SemaphoreType.DMA,
            pltpu.SemaphoreType.DMA,
        ],
    )
    out_flat = call(x, tab_flat)
    return out_flat.reshape(B, T, D)


def kernel(x, tables):
    return _run(x.astype(jnp.int32), tables)
